# baseline (speedup 1.0000x reference)
import functools

import jax
import jax.numpy as jnp
from jax import lax
from jax.experimental import pallas as pl
from jax.experimental.pallas import tpu as pltpu
from jax.experimental.pallas import tpu_sc as plsc

_N_VOCAB = 1000
_D = 1000
_BATCH = 1024
_SEQ = 50
_NC = 2
_NS = 16
_NW = _NC * _NS
_B_TOT = _BATCH * _SEQ
_B_PER_W = _B_TOT // _NW
_C = 40
_NCHUNK = _B_PER_W // _C

_mesh = plsc.VectorSubcoreMesh(core_axis_name="c", subcore_axis_name="s")


@functools.partial(
    pl.kernel,
    mesh=_mesh,
    out_type=jax.ShapeDtypeStruct((_B_TOT, _D), jnp.float32),
    compiler_params=pltpu.CompilerParams(use_tc_tiling_on_sc=False),
    scratch_types=[
        pltpu.VMEM((_NCHUNK, _C), jnp.int32),
        pltpu.VMEM((_C, _D), jnp.float32),
        pltpu.VMEM((_C, _D), jnp.float32),
        pltpu.SemaphoreType.DMA,
        pltpu.SemaphoreType.DMA,
        pltpu.SemaphoreType.DMA,
        pltpu.SemaphoreType.DMA,
    ],
)
def _sc_gather(x_hbm, table_hbm, out_hbm,
               idx, buf0, buf1, sg0, sg1, ss0, ss1):
    wid = lax.axis_index("s") * _NC + lax.axis_index("c")
    base = wid * _B_PER_W

    pltpu.sync_copy(x_hbm.at[wid], idx)

    def gather(i, buf, sem):
        return pltpu.async_copy(table_hbm.at[idx.at[i]], buf, sem)

    def scatter(i, buf, sem):
        return pltpu.async_copy(buf, out_hbm.at[pl.ds(base + i * _C, _C)], sem)

    def drain_scatter(buf, sem):
        pltpu.make_async_copy(buf, out_hbm.at[pl.ds(base, _C)], sem).wait()

    g0 = gather(0, buf0, sg0)
    g1 = gather(1, buf1, sg1)
    g0.wait()
    scatter(0, buf0, ss0)
    g1.wait()
    scatter(1, buf1, ss1)

    def body(c2, carry):
        i0 = 2 * c2
        i1 = i0 + 1
        drain_scatter(buf0, ss0)
        gb0 = gather(i0, buf0, sg0)
        drain_scatter(buf1, ss1)
        gb1 = gather(i1, buf1, sg1)
        gb0.wait()
        scatter(i0, buf0, ss0)
        gb1.wait()
        scatter(i1, buf1, ss1)
        return carry

    lax.fori_loop(1, _NCHUNK // 2, body, 0)

    drain_scatter(buf0, ss0)
    drain_scatter(buf1, ss1)


def kernel(x, table):
    out = _sc_gather(x.reshape(_NW, _NCHUNK, _C), table)
    return out.reshape(_BATCH, _SEQ, _D)


# speedup vs baseline: 1.0005x; 1.0005x over previous
import functools

import jax
import jax.numpy as jnp
from jax import lax
from jax.experimental import pallas as pl
from jax.experimental.pallas import tpu as pltpu
from jax.experimental.pallas import tpu_sc as plsc

_D = 1000
_BATCH = 1024
_SEQ = 50
_NC = 2
_NS = 16
_NW = _NC * _NS
_BT_PER_W = _BATCH // _NW
_C = _SEQ

_mesh = plsc.VectorSubcoreMesh(core_axis_name="c", subcore_axis_name="s")


@functools.partial(
    pl.kernel,
    mesh=_mesh,
    out_type=jax.ShapeDtypeStruct((_BATCH, _SEQ, _D), jnp.float32),
    compiler_params=pltpu.CompilerParams(use_tc_tiling_on_sc=False),
    scratch_types=[
        pltpu.VMEM((_BT_PER_W, _C), jnp.int32),
        pltpu.VMEM((_C, _D), jnp.float32),
        pltpu.VMEM((_C, _D), jnp.float32),
        pltpu.SemaphoreType.DMA,
        pltpu.SemaphoreType.DMA,
        pltpu.SemaphoreType.DMA,
        pltpu.SemaphoreType.DMA,
    ],
)
def _sc_gather(x_hbm, table_hbm, out_hbm,
               idx, buf0, buf1, sg0, sg1, ss0, ss1):
    wid = lax.axis_index("s") * _NC + lax.axis_index("c")
    base = wid * _BT_PER_W

    pltpu.sync_copy(x_hbm.at[pl.ds(base, _BT_PER_W)], idx)

    def gather(i, buf, sem):
        return pltpu.async_copy(table_hbm.at[idx.at[i]], buf, sem)

    def scatter(i, buf, sem):
        return pltpu.async_copy(buf, out_hbm.at[base + i], sem)

    def drain_scatter(buf, sem):
        pltpu.make_async_copy(buf, out_hbm.at[base], sem).wait()

    g0 = gather(0, buf0, sg0)
    g1 = gather(1, buf1, sg1)
    g0.wait()
    scatter(0, buf0, ss0)
    g1.wait()
    scatter(1, buf1, ss1)

    def body(c2, carry):
        i0 = 2 * c2
        i1 = i0 + 1
        drain_scatter(buf0, ss0)
        gb0 = gather(i0, buf0, sg0)
        drain_scatter(buf1, ss1)
        gb1 = gather(i1, buf1, sg1)
        gb0.wait()
        scatter(i0, buf0, ss0)
        gb1.wait()
        scatter(i1, buf1, ss1)
        return carry

    lax.fori_loop(1, _BT_PER_W // 2, body, 0)

    drain_scatter(buf0, ss0)
    drain_scatter(buf1, ss1)


def kernel(x, table):
    return _sc_gather(x, table)


# speedup vs baseline: 2.0305x; 2.0294x over previous
import functools

import jax
import jax.numpy as jnp
from jax import lax
from jax.experimental import pallas as pl
from jax.experimental.pallas import tpu as pltpu
from jax.experimental.pallas import tpu_sc as plsc

_D = 1000
_DP = 1024
_BATCH = 1024
_SEQ = 50
_NC = 2
_NS = 16
_NW = _NC * _NS
_BT_PER_W = _BATCH // _NW

_mesh = plsc.VectorSubcoreMesh(core_axis_name="c", subcore_axis_name="s")


@functools.partial(
    pl.kernel,
    mesh=_mesh,
    out_type=jax.ShapeDtypeStruct((_BATCH, _SEQ, _DP), jnp.float32),
    scratch_types=[
        pltpu.VMEM((_BT_PER_W, _SEQ), jnp.int32),
        pltpu.VMEM((_SEQ, _DP), jnp.float32),
        pltpu.VMEM((_SEQ, _DP), jnp.float32),
        pltpu.SemaphoreType.DMA,
        pltpu.SemaphoreType.DMA,
        pltpu.SemaphoreType.DMA,
        pltpu.SemaphoreType.DMA,
    ],
)
def _sc_gather(x_hbm, table_hbm, out_hbm,
               idx, buf0, buf1, sg0, sg1, ss0, ss1):
    wid = lax.axis_index("s") * _NC + lax.axis_index("c")
    base = wid * _BT_PER_W

    pltpu.sync_copy(x_hbm.at[pl.ds(base, _BT_PER_W)], idx)

    def gather(i, buf, sem):
        return pltpu.async_copy(table_hbm.at[idx.at[i]], buf, sem)

    def scatter(i, buf, sem):
        return pltpu.async_copy(buf, out_hbm.at[base + i], sem)

    def drain_scatter(buf, sem):
        pltpu.make_async_copy(buf, out_hbm.at[base], sem).wait()

    g0 = gather(0, buf0, sg0)
    g1 = gather(1, buf1, sg1)
    g0.wait()
    scatter(0, buf0, ss0)
    g1.wait()
    scatter(1, buf1, ss1)

    def body(c2, carry):
        i0 = 2 * c2
        i1 = i0 + 1
        drain_scatter(buf0, ss0)
        gb0 = gather(i0, buf0, sg0)
        drain_scatter(buf1, ss1)
        gb1 = gather(i1, buf1, sg1)
        gb0.wait()
        scatter(i0, buf0, ss0)
        gb1.wait()
        scatter(i1, buf1, ss1)
        return carry

    lax.fori_loop(1, _BT_PER_W // 2, body, 0)

    drain_scatter(buf0, ss0)
    drain_scatter(buf1, ss1)


def kernel(x, table):
    table_p = jnp.pad(table, ((0, 0), (0, _DP - _D)))
    out_p = _sc_gather(x, table_p)
    return out_p[:, :, :_D]
